# baseline (speedup 1.0000x reference)
import functools

import jax
import jax.numpy as jnp
from jax import lax
from jax.experimental import pallas as pl
from jax.experimental.pallas import tpu as pltpu
from jax.experimental.pallas import tpu_sc as plsc

_K = 1024
_D = 64
_B = 16
_HW = 1024
_N = _B * _HW

_NTILES = 32
_DH = _D // 2
_L = 16


def _argmin_body(x_ref, cb_ref, idx_ref):
    xT = x_ref[0]
    cb = cb_ref[...]
    scoresT = lax.dot_general(cb, xT, (((1,), (0,)), ((), ())),
                              preferred_element_type=jnp.float32)
    x2 = jnp.sum(xT * xT, axis=0, keepdims=True)
    c2 = jnp.sum(cb * cb, axis=1, keepdims=True)
    d2 = (x2 + c2) - 2.0 * scoresT
    m2 = jnp.min(d2, axis=0, keepdims=True)
    m2c = jnp.maximum(m2, 0.0)
    s = jnp.sqrt(m2c)
    hb = lax.bitcast_convert_type(s * s, jnp.int32)
    hi = m2c
    for k in range(-4, 5):
        c = lax.bitcast_convert_type(hb + k, jnp.float32)
        ok = (c >= 0.0) & (jnp.sqrt(c) == s)
        hi = jnp.where(ok, jnp.maximum(hi, c), hi)
    hi = jnp.where(s > 0.0, hi, 0.0)
    kiota = lax.broadcasted_iota(jnp.int32, (_K, _HW), 0)
    idx = jnp.min(jnp.where(d2 <= hi, kiota, _K), axis=0)
    idx_ref[0] = idx.reshape(1, _HW)


def _compute_idx(inp, codebook):
    return pl.pallas_call(
        _argmin_body,
        grid=(_B,),
        in_specs=[
            pl.BlockSpec((1, _D, _HW), lambda b: (b, 0, 0)),
            pl.BlockSpec((_K, _D), lambda b: (0, 0)),
        ],
        out_specs=pl.BlockSpec((1, 1, _HW), lambda b: (b, 0, 0)),
        out_shape=jax.ShapeDtypeStruct((_B, 1, _HW), jnp.int32),
    )(inp, codebook)


@functools.partial(
    pl.kernel,
    mesh=plsc.VectorSubcoreMesh(core_axis_name="c", subcore_axis_name="s"),
    compiler_params=pltpu.CompilerParams(needs_layout_passes=False),
    out_type=jax.ShapeDtypeStruct((_N * _D,), jnp.float32),
    scratch_types=[
        pltpu.VMEM((_K * _D,), jnp.float32),
        pltpu.VMEM((_HW,), jnp.int32),
        pltpu.VMEM((_DH * _HW,), jnp.float32),
        pltpu.SemaphoreType.DMA,
        pltpu.SemaphoreType.DMA,
    ],
)
def _sc_gather(cb_hbm, idx_hbm, out_hbm, cb_v, idx_v, out_v, sem1, sem2):
    wid = lax.axis_index("s") * 2 + lax.axis_index("c")
    b = wid // 2
    dbase = (wid % 2) * _DH
    cp_cb = pltpu.async_copy(cb_hbm, cb_v, sem1)
    cp_idx = pltpu.async_copy(idx_hbm.at[pl.ds(b * _HW, _HW)], idx_v, sem2)
    cp_cb.wait()
    cp_idx.wait()

    def body(g, carry):
        col = g * _L
        base = idx_v[pl.ds(col, _L)] * _D
        for dd in range(_DH):
            out_v[pl.ds(dd * _HW + col, _L)] = plsc.load_gather(
                cb_v, [base + (dbase + dd)])
        return carry

    lax.fori_loop(0, _HW // _L, body, 0)
    pltpu.sync_copy(out_v, out_hbm.at[pl.ds(wid * (_DH * _HW), _DH * _HW)])


def kernel(input, codebook):
    inp = input.reshape(_B, _D, _HW)
    idx3 = _compute_idx(inp, codebook)
    emb_flat = _sc_gather(codebook.reshape(_K * _D), idx3.reshape(_N))
    embed = emb_flat.reshape(_B, _D, 32, 32)
    idxes = idx3.reshape(_B, 32, 32)
    return (embed, idxes)
